# Initial kernel scaffold; baseline (speedup 1.0000x reference)
import jax
import jax.numpy as jnp
from jax.experimental import pallas as pl


def kernel(x1, x2):
    raise NotImplementedError("write your pallas kernel here")


# speedup vs baseline: 1.9725x; 1.9725x over previous
import jax
import jax.numpy as jnp
from jax.experimental import pallas as pl
from jax.experimental.pallas import tpu as pltpu

_WIDTH = 4


def _abcnn2_body(x1_ref, x2_ref, o1_ref, o2_ref):
    bb, m, d = x1_ref.shape
    L = o1_ref.shape[1]
    for g in range(bb):
        a = x1_ref[g]
        b = x2_ref[g]
        gm = jax.lax.dot_general(
            a, b, (((1,), (1,)), ((), ())),
            preferred_element_type=jnp.float32)
        na = jnp.sum(a * a, axis=1, keepdims=True)
        nb = jnp.sum(b * b, axis=1, keepdims=True)
        sq = na + nb.T - 2.0 * gm
        dist = jnp.sqrt(jnp.maximum(sq, 0.0))
        att = 1.0 / (1.0 + dist)
        w_b = jnp.sum(att, axis=1, keepdims=True)
        w_a_row = jnp.sum(att, axis=0, keepdims=True)
        y1 = w_a_row.T * a
        y2 = w_b * b
        o1_ref[g] = (y1[0:L] + y1[1:L + 1] + y1[2:L + 2] + y1[3:L + 3])
        o2_ref[g] = (y2[0:L] + y2[1:L + 1] + y2[2:L + 2] + y2[3:L + 3])


def kernel(x1, x2):
    B, _, M, D = x1.shape
    L = M - (_WIDTH - 1)
    a3 = x1.reshape(B, M, D)
    b3 = x2.reshape(B, M, D)
    BB = 1
    grid = (B // BB,)
    out_sds = jax.ShapeDtypeStruct((B, L, D), x1.dtype)
    w1, w2 = pl.pallas_call(
        _abcnn2_body,
        out_shape=(out_sds, out_sds),
        grid=grid,
        in_specs=[
            pl.BlockSpec((BB, M, D), lambda i: (i, 0, 0)),
            pl.BlockSpec((BB, M, D), lambda i: (i, 0, 0)),
        ],
        out_specs=(
            pl.BlockSpec((BB, L, D), lambda i: (i, 0, 0)),
            pl.BlockSpec((BB, L, D), lambda i: (i, 0, 0)),
        ),
        compiler_params=pltpu.CompilerParams(
            dimension_semantics=("parallel",),
            vmem_limit_bytes=56 * 1024 * 1024,
        ),
        name="abcnn2_attention",
    )(a3, b3)
    return (w1[:, None], w2[:, None])
